# baseline (speedup 1.0000x reference)
import functools

import jax
import jax.numpy as jnp
from jax import lax
from jax.experimental import pallas as pl
from jax.experimental.pallas import tpu as pltpu
from jax.experimental.pallas import tpu_sc as plsc

N_IN = 162
N_OUT = 42
NEIGH = 7
LANES = 16
NUM_CORES = 2
NUM_SUBCORES = 16
NUM_WORKERS = NUM_CORES * NUM_SUBCORES
CHUNK = 256


def _sc_pool(total_rows):
    rows_per_worker = total_rows // NUM_WORKERS
    n_chunks = rows_per_worker // CHUNK
    mesh = plsc.VectorSubcoreMesh(
        core_axis_name="c", subcore_axis_name="s",
        num_cores=NUM_CORES, num_subcores=NUM_SUBCORES)

    @functools.partial(
        pl.kernel,
        out_type=jax.ShapeDtypeStruct((total_rows * N_OUT,), jnp.float32),
        mesh=mesh,
        scratch_types=[
            pltpu.VMEM((CHUNK * N_IN,), jnp.float32),
            pltpu.VMEM((CHUNK * N_OUT,), jnp.float32),
            pltpu.VMEM((N_OUT * NEIGH * LANES,), jnp.int32),
        ],
        compiler_params=pltpu.CompilerParams(needs_layout_passes=False),
    )
    def run(x_hbm, gidx_hbm, out_hbm, in_v, out_v, gidx_v):
        wid = lax.axis_index("s") * NUM_CORES + lax.axis_index("c")
        base_row = wid * rows_per_worker
        pltpu.sync_copy(gidx_hbm, gidx_v)
        oiota = lax.iota(jnp.int32, LANES) * N_OUT

        def chunk_body(g, _):
            row0 = base_row + g * CHUNK
            pltpu.sync_copy(x_hbm.at[pl.ds(row0 * N_IN, CHUNK * N_IN)], in_v)
            for v in range(N_OUT):
                jvecs = [gidx_v[pl.ds((v * NEIGH + j) * LANES, LANES)]
                         for j in range(NEIGH)]

                def row_body(r, _, v=v, jvecs=jvecs):
                    rb = r * (LANES * N_IN)
                    g0 = plsc.load_gather(in_v, [jvecs[0] + rb])
                    g1 = plsc.load_gather(in_v, [jvecs[1] + rb])
                    g2 = plsc.load_gather(in_v, [jvecs[2] + rb])
                    g3 = plsc.load_gather(in_v, [jvecs[3] + rb])
                    g4 = plsc.load_gather(in_v, [jvecs[4] + rb])
                    g5 = plsc.load_gather(in_v, [jvecs[5] + rb])
                    g6 = plsc.load_gather(in_v, [jvecs[6] + rb])
                    s = ((g0 + g1) + (g2 + g3)) + ((g4 + g5) + g6)
                    acc = s * jnp.float32(1.0 / NEIGH)
                    ovec = oiota + (r * (LANES * N_OUT) + v)
                    plsc.store_scatter(out_v, [ovec], acc)
                    return 0

                lax.fori_loop(0, CHUNK // LANES, row_body, 0)
            pltpu.sync_copy(out_v, out_hbm.at[pl.ds(row0 * N_OUT, CHUNK * N_OUT)])
            return 0

        lax.fori_loop(0, n_chunks, chunk_body, 0)

    return run


def kernel(x, down_neigh_indices):
    b, c, n_in = x.shape
    total_rows = b * c
    xf = x.reshape(total_rows * n_in)
    flat_idx = down_neigh_indices.reshape(-1).astype(jnp.int32)
    lanes = jnp.arange(LANES, dtype=jnp.int32)
    gidx = (flat_idx[:, None] + lanes[None, :] * n_in).reshape(-1)
    out_flat = _sc_pool(total_rows)(xf, gidx)
    return out_flat.reshape(b, c, N_OUT)


# speedup vs baseline: 1.1351x; 1.1351x over previous
import functools

import jax
import jax.numpy as jnp
from jax import lax
from jax.experimental import pallas as pl
from jax.experimental.pallas import tpu as pltpu
from jax.experimental.pallas import tpu_sc as plsc

N_IN = 162
N_OUT = 42
NEIGH = 7
LANES = 16
NUM_CORES = 2
NUM_SUBCORES = 16
NUM_WORKERS = NUM_CORES * NUM_SUBCORES
CHUNK = 256
RB_IN = LANES * N_IN
RB_OUT = LANES * N_OUT


def _sc_pool(total_rows):
    rows_per_worker = total_rows // NUM_WORKERS
    n_chunks = rows_per_worker // CHUNK
    mesh = plsc.VectorSubcoreMesh(
        core_axis_name="c", subcore_axis_name="s",
        num_cores=NUM_CORES, num_subcores=NUM_SUBCORES)

    @functools.partial(
        pl.kernel,
        out_type=jax.ShapeDtypeStruct((total_rows * N_OUT,), jnp.float32),
        mesh=mesh,
        scratch_types=[
            pltpu.VMEM((CHUNK * N_IN,), jnp.float32),
            pltpu.VMEM((CHUNK * N_OUT,), jnp.float32),
            pltpu.VMEM((N_OUT * NEIGH * LANES,), jnp.int32),
        ],
        compiler_params=pltpu.CompilerParams(needs_layout_passes=False),
    )
    def run(x_hbm, gidx_hbm, out_hbm, in_v, out_v, gidx_v):
        wid = lax.axis_index("s") * NUM_CORES + lax.axis_index("c")
        base_row = wid * rows_per_worker
        pltpu.sync_copy(gidx_hbm, gidx_v)
        oiota = lax.iota(jnp.int32, LANES) * N_OUT

        def chunk_body(g, _):
            row0 = base_row + g * CHUNK
            pltpu.sync_copy(x_hbm.at[pl.ds(row0 * N_IN, CHUNK * N_IN)], in_v)
            for v in range(N_OUT):
                jvecs = [gidx_v[pl.ds((v * NEIGH + j) * LANES, LANES)]
                         for j in range(NEIGH)]
                ovec = oiota + v

                @plsc.parallel_loop(0, CHUNK // LANES, 1, unroll=2)
                def row_body(r, jvecs=jvecs, ovec=ovec):
                    src = in_v.at[pl.ds(r * RB_IN, RB_IN)]
                    g0 = plsc.load_gather(src, [jvecs[0]])
                    g1 = plsc.load_gather(src, [jvecs[1]])
                    g2 = plsc.load_gather(src, [jvecs[2]])
                    g3 = plsc.load_gather(src, [jvecs[3]])
                    g4 = plsc.load_gather(src, [jvecs[4]])
                    g5 = plsc.load_gather(src, [jvecs[5]])
                    g6 = plsc.load_gather(src, [jvecs[6]])
                    s = ((g0 + g1) + (g2 + g3)) + ((g4 + g5) + g6)
                    acc = s * jnp.float32(1.0 / NEIGH)
                    dst = out_v.at[pl.ds(r * RB_OUT, RB_OUT)]
                    plsc.store_scatter(dst, [ovec], acc)

            pltpu.sync_copy(out_v, out_hbm.at[pl.ds(row0 * N_OUT, CHUNK * N_OUT)])
            return 0

        lax.fori_loop(0, n_chunks, chunk_body, 0)

    return run


def kernel(x, down_neigh_indices):
    b, c, n_in = x.shape
    total_rows = b * c
    xf = x.reshape(total_rows * n_in)
    flat_idx = down_neigh_indices.reshape(-1).astype(jnp.int32)
    lanes = jnp.arange(LANES, dtype=jnp.int32)
    gidx = (flat_idx[:, None] + lanes[None, :] * n_in).reshape(-1)
    out_flat = _sc_pool(total_rows)(xf, gidx)
    return out_flat.reshape(b, c, N_OUT)


# speedup vs baseline: 2.6499x; 2.3346x over previous
import functools
import jax
import jax.numpy as jnp
from jax import lax
from jax.experimental import pallas as pl
from jax.experimental.pallas import tpu as pltpu

N_IN = 162
N_OUT = 42
NEIGH = 7
BLK = 2048


def _pool_body(idx_ref, x_ref, o_ref):
    rowi = lax.broadcasted_iota(jnp.int32, (N_IN, N_OUT), 0)
    m = jnp.zeros((N_IN, N_OUT), dtype=jnp.float32)
    for j in range(NEIGH):
        idx_j = idx_ref[j:j + 1, :]
        m = m + jnp.where(rowi == idx_j, jnp.float32(1.0 / NEIGH),
                          jnp.float32(0.0))
    o_ref[:, :] = jnp.dot(x_ref[:, :], m,
                          preferred_element_type=jnp.float32)


def _tc_pool(total_rows):
    grid = total_rows // BLK
    return pl.pallas_call(
        _pool_body,
        grid=(grid,),
        in_specs=[
            pl.BlockSpec((NEIGH, N_OUT), lambda i: (0, 0)),
            pl.BlockSpec((BLK, N_IN), lambda i: (i, 0)),
        ],
        out_specs=pl.BlockSpec((BLK, N_OUT), lambda i: (i, 0)),
        out_shape=jax.ShapeDtypeStruct((total_rows, N_OUT), jnp.float32),
    )


def kernel(x, down_neigh_indices):
    b, c, n_in = x.shape
    total_rows = b * c
    xf = x.reshape(total_rows, n_in)
    idx_t = down_neigh_indices.astype(jnp.int32).T
    out = _tc_pool(total_rows)(idx_t, xf)
    return out.reshape(b, c, N_OUT)
